# baseline (speedup 1.0000x reference)
import numpy as np
import jax
import jax.numpy as jnp
from jax.experimental import pallas as pl
from jax.experimental.pallas import tpu as pltpu

_B, _C, _H, _W = 32, 256, 56, 56
_NH = 8
_D = 32
_WS = 7
_S = _WS * _WS
_NWW = _W // _WS
_ROW = _WS * _W
_WPAD = 64
_ROWP = _NWW * _WPAD
_CLAMP_MAX = 4.6052
_EPS = 1e-12
_NEG = -1e30


def _perm_np():
    p = np.zeros((_ROW, _ROWP), dtype=np.float32)
    for i1 in range(_WS):
        for ww in range(_NWW):
            for i2 in range(_WS):
                p[i1 * _W + ww * _WS + i2, ww * _WPAD + i1 * _WS + i2] = 1.0
    return p


def _mask_np():
    r = np.arange(128)[:, None]
    p = np.arange(128)[None, :]
    ok = ((r // _WPAD) == (p // _WPAD)) & ((r % _WPAD) < _S)
    return np.where(ok, 0.0, _NEG).astype(np.float32)


def _attn_kernel(sc_ref, q_ref, k_ref, v_ref, p_ref, pt_ref, lm_ref, o_ref):
    q2 = q_ref[0, :, 0, 0]
    k2 = k_ref[0, :, 0, 0]
    v2 = v_ref[0, :, 0, 0]

    def l2n(x):
        x3 = x.reshape(_NH, _D, _ROW)
        n = jnp.sqrt(jnp.sum(x3 * x3, axis=1, keepdims=True))
        return (x3 / jnp.maximum(n, _EPS)).reshape(_C, _ROW)

    pm = p_ref[...]
    pt = pt_ref[...]
    kn = l2n(k2)
    qw = jnp.dot(l2n(q2), pm, preferred_element_type=jnp.float32)
    vw = jnp.dot(v2, pm, preferred_element_type=jnp.float32)

    lm = lm_ref[...]
    head_rows = []
    for h in range(_NH):
        sc = sc_ref[h]
        qh = qw[h * _D:(h + 1) * _D, :] * sc
        kh_t = jax.lax.dot_general(
            pt, kn[h * _D:(h + 1) * _D, :], (((1,), (1,)), ((), ())),
            preferred_element_type=jnp.float32)
        vh = vw[h * _D:(h + 1) * _D, :]
        lmh = lm - sc
        chunks = []
        for c in range(4):
            sl = slice(c * 128, (c + 1) * 128)
            st = jax.lax.dot_general(
                kh_t[sl, :], qh[:, sl], (((1,), (0,)), ((), ())),
                preferred_element_type=jnp.float32)
            e = jnp.exp(st + lmh)
            den = jnp.sum(e, axis=0, keepdims=True)
            o_c = jax.lax.dot_general(
                vh[:, sl], e, (((1,), (0,)), ((), ())),
                preferred_element_type=jnp.float32)
            chunks.append(o_c / den)
        head_rows.append(jnp.concatenate(chunks, axis=1))
    outw = jnp.concatenate(head_rows, axis=0)
    o_ref[0, :, 0, 0] = jnp.dot(outw, pt_ref[...],
                                preferred_element_type=jnp.float32)


def kernel(q, k, v, logit_scale):
    sc = jnp.exp(jnp.minimum(logit_scale, _CLAMP_MAX)).reshape(_NH)
    pm = jnp.asarray(_perm_np())
    pt = pm.T
    lm = jnp.asarray(_mask_np())

    nr = _H // _WS
    q5 = q.reshape(_B, _C, nr, 1, _ROW)
    k5 = k.reshape(_B, _C, nr, 1, _ROW)
    v5 = v.reshape(_B, _C, nr, 1, _ROW)

    strip = pl.BlockSpec((1, _C, 1, 1, _ROW), lambda b, r: (b, 0, r, 0, 0))
    fixed = lambda shape: pl.BlockSpec(shape, lambda b, r: tuple([0] * len(shape)))
    out = pl.pallas_call(
        _attn_kernel,
        out_shape=jax.ShapeDtypeStruct((_B, _C, nr, 1, _ROW), jnp.float32),
        grid=(_B, nr),
        in_specs=[pl.BlockSpec(memory_space=pltpu.SMEM),
                  strip, strip, strip,
                  fixed((_ROW, _ROWP)), fixed((_ROWP, _ROW)),
                  fixed((128, 128))],
        out_specs=strip,
        compiler_params=pltpu.CompilerParams(
            dimension_semantics=("parallel", "arbitrary")),
        name="win_cos_attn",
    )(sc, q5, k5, v5, pm, pt, lm)
    return out.reshape(_B, _C, _H, _W)


# speedup vs baseline: 1.1640x; 1.1640x over previous
import numpy as np
import jax
import jax.numpy as jnp
from jax.experimental import pallas as pl
from jax.experimental.pallas import tpu as pltpu

_B, _C, _H, _W = 32, 256, 56, 56
_NH = 8
_D = 32
_WS = 7
_S = _WS * _WS
_NWW = _W // _WS
_ROW = _WS * _W
_WPAD = 64
_ROWP = _NWW * _WPAD
_CLAMP_MAX = 4.6052
_EPS = 1e-12
_NEG = -1e30


def _perm_np():
    p = np.zeros((_ROW, _ROWP), dtype=np.float32)
    for i1 in range(_WS):
        for ww in range(_NWW):
            for i2 in range(_WS):
                p[i1 * _W + ww * _WS + i2, ww * _WPAD + i1 * _WS + i2] = 1.0
    return p


def _mask_np():
    r = np.arange(128)[:, None]
    p = np.arange(128)[None, :]
    ok = ((r // _WPAD) == (p // _WPAD)) & ((r % _WPAD) < _S)
    return np.where(ok, 0.0, _NEG).astype(np.float32)


def _attn_kernel(sc_ref, q_ref, k_ref, v_ref, p_ref, pt_ref, lm_ref, o_ref):
    q2 = q_ref[0, :, 0, 0]
    k2 = k_ref[0, :, 0, 0]
    v2 = v_ref[0, :, 0, 0]

    def l2n(x):
        x3 = x.reshape(_NH, _D, _ROW)
        n = jnp.sqrt(jnp.sum(x3 * x3, axis=1, keepdims=True))
        return (x3 / jnp.maximum(n, _EPS)).reshape(_C, _ROW)

    pm = p_ref[...]
    kw = jnp.dot(l2n(k2), pm, preferred_element_type=jnp.float32)
    qw = jnp.dot(l2n(q2), pm, preferred_element_type=jnp.float32)
    vw = jnp.dot(v2, pm, preferred_element_type=jnp.float32)

    lm = lm_ref[...]
    head_rows = []
    for h in range(_NH):
        sc = sc_ref[h]
        qh = qw[h * _D:(h + 1) * _D, :] * sc
        kh_t = jnp.transpose(kw[h * _D:(h + 1) * _D, :])
        vh = vw[h * _D:(h + 1) * _D, :]
        lmh = lm - sc
        chunks = []
        for c in range(4):
            sl = slice(c * 128, (c + 1) * 128)
            st = jax.lax.dot_general(
                kh_t[sl, :], qh[:, sl], (((1,), (0,)), ((), ())),
                preferred_element_type=jnp.float32)
            e = jnp.exp(st + lmh)
            den = jnp.sum(e, axis=0, keepdims=True)
            o_c = jax.lax.dot_general(
                vh[:, sl], e, (((1,), (0,)), ((), ())),
                preferred_element_type=jnp.float32)
            chunks.append(o_c / den)
        head_rows.append(jnp.concatenate(chunks, axis=1))
    outw = jnp.concatenate(head_rows, axis=0)
    o_ref[0, :, 0, 0] = jnp.dot(outw, pt_ref[...],
                                preferred_element_type=jnp.float32)


def kernel(q, k, v, logit_scale):
    sc = jnp.exp(jnp.minimum(logit_scale, _CLAMP_MAX)).reshape(_NH)
    pm = jnp.asarray(_perm_np())
    pt = pm.T
    lm = jnp.asarray(_mask_np())

    nr = _H // _WS
    q5 = q.reshape(_B, _C, nr, 1, _ROW)
    k5 = k.reshape(_B, _C, nr, 1, _ROW)
    v5 = v.reshape(_B, _C, nr, 1, _ROW)

    strip = pl.BlockSpec((1, _C, 1, 1, _ROW), lambda b, r: (b, 0, r, 0, 0))
    fixed = lambda shape: pl.BlockSpec(shape, lambda b, r: tuple([0] * len(shape)))
    out = pl.pallas_call(
        _attn_kernel,
        out_shape=jax.ShapeDtypeStruct((_B, _C, nr, 1, _ROW), jnp.float32),
        grid=(_B, nr),
        in_specs=[pl.BlockSpec(memory_space=pltpu.SMEM),
                  strip, strip, strip,
                  fixed((_ROW, _ROWP)), fixed((_ROWP, _ROW)),
                  fixed((128, 128))],
        out_specs=strip,
        compiler_params=pltpu.CompilerParams(
            dimension_semantics=("parallel", "arbitrary")),
        name="win_cos_attn",
    )(sc, q5, k5, v5, pm, pt, lm)
    return out.reshape(_B, _C, _H, _W)


# speedup vs baseline: 1.2747x; 1.0951x over previous
import numpy as np
import jax
import jax.numpy as jnp
from jax.experimental import pallas as pl
from jax.experimental.pallas import tpu as pltpu

_B, _C, _H, _W = 32, 256, 56, 56
_NH = 8
_D = 32
_WS = 7
_S = _WS * _WS
_NWW = _W // _WS
_ROW = _WS * _W
_WPAD = 64
_ROWP = _NWW * _WPAD
_CLAMP_MAX = 4.6052
_EPS = 1e-12
_NEG = -1e30


def _shift_np():
    f = np.zeros((_WS, _W, _ROWP), dtype=np.float32)
    for i1 in range(_WS):
        for ww in range(_NWW):
            for i2 in range(_WS):
                f[i1, ww * _WS + i2, ww * _WPAD + i1 * _WS + i2] = 1.0
    return f


def _pt_np():
    pt = np.zeros((_ROWP, _ROW), dtype=np.float32)
    for i1 in range(_WS):
        for ww in range(_NWW):
            for i2 in range(_WS):
                pt[ww * _WPAD + i1 * _WS + i2, i1 * _W + ww * _WS + i2] = 1.0
    return pt


def _mask_np():
    r = np.arange(128)[:, None]
    p = np.arange(128)[None, :]
    ok = ((r // _WPAD) == (p // _WPAD)) & ((r % _WPAD) < _S)
    return np.where(ok, 0.0, _NEG).astype(np.float32)


def _attn_kernel(sc_ref, q_ref, k_ref, v_ref, f_ref, pt_ref, lm_ref, o_ref):
    r = pl.program_id(1)

    def assemble(ref):
        acc = None
        for i1 in range(_WS):
            row = ref[0, :, _WS * r + i1, :]
            part = jnp.dot(row, f_ref[i1], preferred_element_type=jnp.float32)
            acc = part if acc is None else acc + part
        return acc

    def l2n(x):
        x3 = x.reshape(_NH, _D, _ROWP)
        n = jnp.sqrt(jnp.sum(x3 * x3, axis=1, keepdims=True))
        return (x3 / jnp.maximum(n, _EPS)).reshape(_C, _ROWP)

    qw = l2n(assemble(q_ref))
    kw = l2n(assemble(k_ref))
    vw = assemble(v_ref)

    lm = lm_ref[...]
    head_rows = []
    for h in range(_NH):
        sc = sc_ref[h]
        qh = qw[h * _D:(h + 1) * _D, :] * sc
        kh_t = jnp.transpose(kw[h * _D:(h + 1) * _D, :])
        vh = vw[h * _D:(h + 1) * _D, :]
        lmh = lm - sc
        chunks = []
        for c in range(4):
            sl = slice(c * 128, (c + 1) * 128)
            st = jax.lax.dot_general(
                kh_t[sl, :], qh[:, sl], (((1,), (0,)), ((), ())),
                preferred_element_type=jnp.float32)
            e = jnp.exp(st + lmh)
            den = jnp.sum(e, axis=0, keepdims=True)
            o_c = jax.lax.dot_general(
                vh[:, sl], e, (((1,), (0,)), ((), ())),
                preferred_element_type=jnp.float32)
            chunks.append(o_c / den)
        head_rows.append(jnp.concatenate(chunks, axis=1))
    outw = jnp.concatenate(head_rows, axis=0)
    o_ref[0, :, 0, 0] = jnp.dot(outw, pt_ref[...],
                                preferred_element_type=jnp.float32)


def kernel(q, k, v, logit_scale):
    sc = jnp.exp(jnp.minimum(logit_scale, _CLAMP_MAX)).reshape(_NH)
    fm = jnp.asarray(_shift_np())
    pt = jnp.asarray(_pt_np())
    lm = jnp.asarray(_mask_np())

    nr = _H // _WS
    img = pl.BlockSpec((1, _C, _H, _W), lambda b, r: (b, 0, 0, 0))
    fixed = lambda shape: pl.BlockSpec(shape, lambda b, r: tuple([0] * len(shape)))
    out = pl.pallas_call(
        _attn_kernel,
        out_shape=jax.ShapeDtypeStruct((_B, _C, nr, 1, _ROW), jnp.float32),
        grid=(_B, nr),
        in_specs=[pl.BlockSpec(memory_space=pltpu.SMEM),
                  img, img, img,
                  fixed((_WS, _W, _ROWP)), fixed((_ROWP, _ROW)),
                  fixed((128, 128))],
        out_specs=pl.BlockSpec((1, _C, 1, 1, _ROW), lambda b, r: (b, 0, r, 0, 0)),
        compiler_params=pltpu.CompilerParams(
            dimension_semantics=("parallel", "arbitrary"),
            vmem_limit_bytes=100 * 1024 * 1024),
        name="win_cos_attn",
    )(sc, q, k, v, fm, pt, lm)
    return out.reshape(_B, _C, _H, _W)
